# Initial kernel scaffold; baseline (speedup 1.0000x reference)
import jax
import jax.numpy as jnp
from jax.experimental import pallas as pl


def kernel(x, embedding, Wl1, Wr1, att1, b1, Wl2, Wr2, att2, b2):
    raise NotImplementedError("write your pallas kernel here")


# speedup vs baseline: 881.0521x; 881.0521x over previous
import jax
import jax.numpy as jnp
from jax import lax
from jax.experimental import pallas as pl
from jax.experimental.pallas import tpu as pltpu

N = 256
D = 128
HEADS = 4
DH = D // HEADS
NEG_INF = float("-inf")


def _gat_layer(x, wl, wr, att_ref, bias, adj):
    xl = lax.dot_general(x, wl, (((1,), (0,)), ((), ())),
                         preferred_element_type=jnp.float32)
    xr = lax.dot_general(x, wr, (((1,), (0,)), ((), ())),
                         preferred_element_type=jnp.float32)
    xlt = xl.T
    outs = []
    for h in range(HEADS):
        acc = jnp.zeros((N, N), jnp.float32)
        for k in range(DH):
            c = h * DH + k
            col = xr[:, c:c + 1]
            row = xlt[c:c + 1, :]
            z = col + row
            acc = acc + jnp.maximum(z, 0.2 * z) * att_ref[h, k]
        e = jnp.where(adj != 0.0, acc, NEG_INF)
        m = jnp.max(e, axis=1, keepdims=True)
        m = jnp.where(jnp.isfinite(m), m, 0.0)
        ex = jnp.exp(e - m)
        denom = jnp.sum(ex, axis=1, keepdims=True)
        alpha = ex / (denom + 1e-16)
        outs.append(lax.dot_general(
            alpha, xl[:, h * DH:(h + 1) * DH],
            (((1,), (0,)), ((), ())),
            preferred_element_type=jnp.float32))
    return jnp.concatenate(outs, axis=1) + bias


def _body(x_ref, emb_ref, wl1_ref, wr1_ref, b1_ref, wl2_ref, wr2_ref,
          b2_ref, att1_ref, att2_ref, out_ref):
    x = x_ref[0]
    emb = emb_ref[...]
    sq = jnp.sum(emb * emb, axis=1, keepdims=True)
    nrm = jnp.maximum(jnp.sqrt(sq), 1e-12)
    ne = emb / nrm
    adj = lax.dot_general(ne, ne, (((1,), (1,)), ((), ())),
                          preferred_element_type=jnp.float32)
    h1 = _gat_layer(x, wl1_ref[...], wr1_ref[...], att1_ref, b1_ref[...], adj)
    h1 = jnp.maximum(h1, 0.0)
    out_ref[0] = _gat_layer(h1, wl2_ref[...], wr2_ref[...], att2_ref,
                            b2_ref[...], adj)


def kernel(x, embedding, Wl1, Wr1, att1, b1, Wl2, Wr2, att2, b2):
    batch = x.shape[0]
    full = lambda shape: pl.BlockSpec(shape, lambda b: (0,) * len(shape))
    out = pl.pallas_call(
        _body,
        grid=(batch,),
        in_specs=[
            pl.BlockSpec((1, N, D), lambda b: (b, 0, 0)),
            full((N, D)),
            full((D, D)),
            full((D, D)),
            full((1, D)),
            full((D, D)),
            full((D, D)),
            full((1, D)),
            pl.BlockSpec(memory_space=pltpu.SMEM),
            pl.BlockSpec(memory_space=pltpu.SMEM),
        ],
        out_specs=pl.BlockSpec((1, N, D), lambda b: (b, 0, 0)),
        out_shape=jax.ShapeDtypeStruct((batch, N, D), jnp.float32),
    )(x, embedding, Wl1, Wr1, b1.reshape(1, D), Wl2, Wr2,
      b2.reshape(1, D), att1, att2)
    return out


# speedup vs baseline: 1257.4885x; 1.4273x over previous
import jax
import jax.numpy as jnp
from jax import lax
from jax.experimental import pallas as pl
from jax.experimental.pallas import tpu as pltpu

N = 256
D = 128
HEADS = 4
DH = D // HEADS
NEG_INF = float("-inf")


def _gat_layer(x, wl, wr, att_ref, att06_ref, bias, adj):
    xl = lax.dot_general(x, wl, (((1,), (0,)), ((), ())),
                         preferred_element_type=jnp.float32)
    xr = lax.dot_general(x, wr, (((1,), (0,)), ((), ())),
                         preferred_element_type=jnp.float32)
    xlt = xl.T
    xrb = xr.astype(jnp.bfloat16)
    xltb = xlt.astype(jnp.bfloat16)
    outs = []
    for h in range(HEADS):
        xl_h = xl[:, h * DH:(h + 1) * DH]
        xr_h = xr[:, h * DH:(h + 1) * DH]
        a06 = att06_ref[h:h + 1, :]
        sl_row = lax.dot_general(a06, xl_h, (((1,), (1,)), ((), ())),
                                 preferred_element_type=jnp.float32)
        sr_col = lax.dot_general(xr_h, a06, (((1,), (1,)), ((), ())),
                                 preferred_element_type=jnp.float32)
        acc = jnp.zeros((N, N), jnp.bfloat16)
        for k in range(DH):
            c = h * DH + k
            col = xrb[:, c:c + 1]
            row = xltb[c:c + 1, :]
            z = col + row
            s_k = (att_ref[h, k] * 0.4).astype(jnp.bfloat16)
            acc = acc + jnp.abs(z) * s_k
        e0 = (sr_col + sl_row) + acc.astype(jnp.float32)
        e = jnp.where(adj != 0.0, e0, NEG_INF)
        m = jnp.max(e, axis=1, keepdims=True)
        m = jnp.where(jnp.isfinite(m), m, 0.0)
        ex = jnp.exp(e - m)
        denom = jnp.sum(ex, axis=1, keepdims=True)
        alpha = ex / (denom + 1e-16)
        outs.append(lax.dot_general(
            alpha, xl[:, h * DH:(h + 1) * DH],
            (((1,), (0,)), ((), ())),
            preferred_element_type=jnp.float32))
    return jnp.concatenate(outs, axis=1) + bias


def _body(x_ref, emb_ref, wl1_ref, wr1_ref, b1_ref, wl2_ref, wr2_ref,
          b2_ref, att1v_ref, att2v_ref, att1_ref, att2_ref, out_ref):
    x = x_ref[0]
    emb = emb_ref[...]
    sq = jnp.sum(emb * emb, axis=1, keepdims=True)
    nrm = jnp.maximum(jnp.sqrt(sq), 1e-12)
    ne = emb / nrm
    adj = lax.dot_general(ne, ne, (((1,), (1,)), ((), ())),
                          preferred_element_type=jnp.float32)
    h1 = _gat_layer(x, wl1_ref[...], wr1_ref[...], att1_ref, att1v_ref[...],
                    b1_ref[...], adj)
    h1 = jnp.maximum(h1, 0.0)
    out_ref[0] = _gat_layer(h1, wl2_ref[...], wr2_ref[...], att2_ref,
                            att2v_ref[...], b2_ref[...], adj)


def kernel(x, embedding, Wl1, Wr1, att1, b1, Wl2, Wr2, att2, b2):
    batch = x.shape[0]
    full = lambda shape: pl.BlockSpec(shape, lambda b: (0,) * len(shape))
    out = pl.pallas_call(
        _body,
        grid=(batch,),
        in_specs=[
            pl.BlockSpec((1, N, D), lambda b: (b, 0, 0)),
            full((N, D)),
            full((D, D)),
            full((D, D)),
            full((1, D)),
            full((D, D)),
            full((D, D)),
            full((1, D)),
            full((HEADS, DH)),
            full((HEADS, DH)),
            pl.BlockSpec(memory_space=pltpu.SMEM),
            pl.BlockSpec(memory_space=pltpu.SMEM),
        ],
        out_specs=pl.BlockSpec((1, N, D), lambda b: (b, 0, 0)),
        out_shape=jax.ShapeDtypeStruct((batch, N, D), jnp.float32),
    )(x, embedding, Wl1, Wr1, b1.reshape(1, D), Wl2, Wr2,
      b2.reshape(1, D), 0.6 * att1, 0.6 * att2, att1, att2)
    return out
